# baseline (speedup 1.0000x reference)
import jax
import jax.numpy as jnp
from jax.experimental import pallas as pl
from jax.experimental.pallas import tpu as pltpu

H, W_ = 180, 360
HW = H * W_
_BC = 32


def _fold_body(x_ref, o_hbm, buf, sems):
    n = pl.program_id(0)
    cb = pl.program_id(1)
    ncb = pl.num_programs(1)
    s = n * ncb + cb
    slot = s % 2
    last = pl.num_programs(0) * ncb - 1

    buf[slot] = x_ref[0].reshape(_BC, H, W_)
    cur = pltpu.make_async_copy(
        buf.at[slot], o_hbm.at[n, pl.ds(cb * _BC, _BC)], sems.at[slot])
    cur.start()

    @pl.when(s > 0)
    def _wait_prev():
        ps = s - 1
        pn = ps // ncb
        pcb = ps - pn * ncb
        pltpu.make_async_copy(
            buf.at[slot ^ 1],
            o_hbm.at[pn, pl.ds(pcb * _BC, _BC)],
            sems.at[slot ^ 1],
        ).wait()

    @pl.when(s == last)
    def _wait_last():
        cur.wait()


def kernel(x):
    N, C, L = x.shape
    out = pl.pallas_call(
        _fold_body,
        grid=(N, C // _BC),
        in_specs=[pl.BlockSpec((1, _BC, L), lambda n, c: (n, c, 0))],
        out_specs=pl.BlockSpec(memory_space=pl.ANY),
        out_shape=jax.ShapeDtypeStruct((N, C, H, W_), x.dtype),
        scratch_shapes=[
            pltpu.VMEM((2, _BC, H, W_), jnp.float32),
            pltpu.SemaphoreType.DMA((2,)),
        ],
    )(x)
    return out
